# baseline (speedup 1.0000x reference)
import functools

import numpy as np
import jax
import jax.numpy as jnp
from jax import lax
from jax.experimental import pallas as pl
from jax.experimental.pallas import tpu as pltpu
from jax.experimental.pallas import tpu_sc as plsc

N = 512
NK = 2 * (N - 1)
NKP = 1024
NSEG = 426
NSEGP = 512
NF = 16
NFH = 8
EPS = 1e-8
DUMP = N * N
QROWS = 128
QWORDS = QROWS * N

_f = np.fft.fftfreq(N).astype(np.float64)
_c = np.fft.fft(_f * _f).real
_i = np.arange(N)
_M_NP = _c[(_i[:, None] - _i[None, :]) % N].astype(np.float32)
_M_HI_NP = _M_NP.astype(np.dtype("bfloat16"))
_M_LO_NP = (_M_NP - _M_HI_NP.astype(np.float32)).astype(np.dtype("bfloat16"))


def _coords_body(p1x_r, p1y_r, p2x_r, p2y_r, segf_r, out_ref):
    p1x, p1y, p2x, p2y, segf = (
        p1x_r[0], p1y_r[0], p2x_r[0], p2y_r[0], segf_r[0],
    )
    x1 = jnp.floor(p1x * (N - 1)).astype(jnp.int32)
    y1 = jnp.floor(p1y * (N - 1)).astype(jnp.int32)
    x2 = jnp.floor(p2x * (N - 1)).astype(jnp.int32)
    y2 = jnp.floor(p2y * (N - 1)).astype(jnp.int32)
    in01 = (
        (p1x >= 0) & (p1x <= 1) & (p1y >= 0) & (p1y <= 1)
        & (p2x >= 0) & (p2x <= 1) & (p2y >= 0) & (p2y <= 1)
    )
    ok = in01 & (segf > 0)
    d = jnp.maximum(2 * jnp.maximum(jnp.abs(x2 - x1), jnp.abs(y2 - y1)), 2) - 1
    d2 = 2 * d
    rec = 1.0 / d2.astype(jnp.float32)
    k = lax.broadcasted_iota(jnp.int32, (NKP, NSEGP), 0)
    kk = jnp.minimum(k, d)

    def interp(a1, a2):
        num = 2 * (a1 * (d - kk) + a2 * kk) + d
        q0 = jnp.floor(num.astype(jnp.float32) * rec).astype(jnp.int32)
        r = num - q0 * d2
        return q0 + (r >= d2).astype(jnp.int32) - (r < 0).astype(jnp.int32)

    lx = interp(x1, x2)
    ly = interp(y1, y2)
    lin = ly * N + lx
    out_ref[0] = jnp.where(ok, lin, DUMP)


def _coords(p1x, p1y, p2x, p2y, segf):
    spec = pl.BlockSpec((1, 1, NSEGP), lambda i: (i, 0, 0))
    return pl.pallas_call(
        _coords_body,
        grid=(NFH,),
        in_specs=[spec] * 5,
        out_specs=pl.BlockSpec((1, NKP, NSEGP), lambda i: (i, 0, 0)),
        out_shape=jax.ShapeDtypeStruct((NFH, NKP, NSEGP), jnp.int32),
    )(p1x, p1y, p2x, p2y, segf)


KROWS = 32
NCH = NKP // KROWS
NSEG_SCAN = 432
JV = NSEG_SCAN // 16


def _scatter_fields(idx):
    mesh = plsc.VectorSubcoreMesh(core_axis_name="c", subcore_axis_name="s")

    @functools.partial(
        pl.kernel,
        mesh=mesh,
        out_type=jax.ShapeDtypeStruct((NFH, 4, QWORDS), jnp.float32),
        scratch_types=[
            pltpu.VMEM((KROWS, NSEGP), jnp.int32),
            pltpu.VMEM((KROWS, NSEGP), jnp.int32),
            pltpu.VMEM((QWORDS,), jnp.float32),
            pltpu.SemaphoreType.DMA,
            pltpu.SemaphoreType.DMA,
        ],
        compiler_params=pltpu.CompilerParams(needs_layout_passes=False),
    )
    def scat(idx_hbm, out_hbm, buf0, buf1, field, sem0, sem1):
        wid = lax.axis_index("s") * 2 + lax.axis_index("c")
        ones = jnp.full((16,), 1.0, jnp.float32)
        zeros16 = jnp.zeros((16,), jnp.float32)
        qwords_u = jnp.uint32(QWORDS)
        fld = wid // 4
        base = (wid % 4) * QWORDS

        def src(c):
            return idx_hbm.at[fld, pl.ds(c * KROWS, KROWS)]

        pltpu.async_copy(src(0), buf0, sem0)
        pltpu.async_copy(src(1), buf1, sem1)

        @plsc.parallel_loop(0, QWORDS // 16, unroll=8)
        def _(i):
            field[pl.ds(i * 16, 16)] = zeros16

        def do_chunk(c, buf, sem):
            pltpu.make_async_copy(src(c), buf, sem).wait()
            for r in range(KROWS):
                @plsc.parallel_loop(0, JV, unroll=9)
                def _(j):
                    idxv = buf[r, pl.ds(j * 16, 16)]
                    loc = idxv - base
                    msk = plsc.bitcast(loc, jnp.uint32) < qwords_u
                    plsc.store_scatter(field, [loc], ones, mask=msk)

            @pl.when(c + 2 < NCH)
            def _():
                pltpu.async_copy(src(c + 2), buf, sem)

        def pair_body(t, _):
            do_chunk(2 * t, buf0, sem0)
            do_chunk(2 * t + 1, buf1, sem1)
            return 0

        lax.fori_loop(0, NCH // 2, pair_body, 0)
        pltpu.sync_copy(field, out_hbm.at[fld, wid % 4])

    return scat(idx)


def _dilate3(x):
    z_row = jnp.zeros((1, N), jnp.float32)
    up = jnp.concatenate([x[1:], z_row], axis=0)
    dn = jnp.concatenate([z_row, x[:-1]], axis=0)
    v = jnp.maximum(x, jnp.maximum(up, dn))
    z_col = jnp.zeros((N, 1), jnp.float32)
    lf = jnp.concatenate([v[:, 1:], z_col], axis=1)
    rt = jnp.concatenate([z_col, v[:, :-1]], axis=1)
    return jnp.maximum(v, jnp.maximum(lf, rt))


def _loss_body(gt_ref, pr_ref, mhi_ref, out_ref):
    b = pl.program_id(0)
    g = _dilate3(gt_ref[0])
    p = _dilate3(pr_ref[0])
    dd = g - p
    ddb = dd.astype(jnp.bfloat16)
    mhi = mhi_ref[...]
    q = jnp.dot(mhi, ddb, preferred_element_type=jnp.float32) + jnp.dot(
        ddb, mhi, preferred_element_type=jnp.float32
    )
    part = jnp.float32(N) * jnp.sum(q * dd) + jnp.float32(EPS * N * N) * jnp.sum(dd * dd)
    part = part * jnp.float32(1.0 / (float(N) ** 4) / 8.0)

    @pl.when(b == 0)
    def _():
        out_ref[0, 0] = 0.0

    out_ref[0, 0] += part


def _spectral_loss(fields, mhi):
    return pl.pallas_call(
        _loss_body,
        grid=(NFH // 2,),
        in_specs=[
            pl.BlockSpec((1, N, N), lambda b: (2 * b, 0, 0)),
            pl.BlockSpec((1, N, N), lambda b: (2 * b + 1, 0, 0)),
            pl.BlockSpec((N, N), lambda b: (0, 0)),
        ],
        out_specs=pl.BlockSpec(memory_space=pltpu.SMEM),
        out_shape=jax.ShapeDtypeStruct((1, 1), jnp.float32),
    )(fields, fields, mhi)


def kernel(pred_keypoints, gt_keypoints, valid_mask):
    kp = jnp.stack([gt_keypoints, pred_keypoints], axis=1).reshape(NF, 6, 72, 2)
    vm = jnp.repeat(valid_mask, 2, axis=0)

    p1 = kp[:, :, :-1, :].reshape(NF, NSEG, 2)
    p2 = kp[:, :, 1:, :].reshape(NF, NSEG, 2)
    segv = vm[:, :, :-1] & vm[:, :, 1:] & jnp.any(vm, axis=2)[:, :, None]
    segf = segv.reshape(NF, NSEG).astype(jnp.float32)

    pad = NSEGP - NSEG

    def padded(a):
        return jnp.pad(a, ((0, 0), (0, pad)))[:, None, :]

    p1x = padded(p1[:, :, 0])
    p1y = padded(p1[:, :, 1])
    p2x = padded(p2[:, :, 0])
    p2y = padded(p2[:, :, 1])
    segf = padded(segf)

    mhi = jnp.asarray(_M_HI_NP)

    loss = jnp.float32(0.0)
    for h in range(2):
        sl = slice(h * NFH, (h + 1) * NFH)
        idx_h = _coords(p1x[sl], p1y[sl], p2x[sl], p2y[sl], segf[sl])
        quarters_h = _scatter_fields(idx_h)
        fields_h = quarters_h.reshape(NFH, N, N)
        out_h = _spectral_loss(fields_h, mhi)
        loss = loss + out_h[0, 0]
    return loss


# speedup vs baseline: 1.3414x; 1.3414x over previous
import functools

import numpy as np
import jax
import jax.numpy as jnp
from jax import lax
from jax.experimental import pallas as pl
from jax.experimental.pallas import tpu as pltpu
from jax.experimental.pallas import tpu_sc as plsc

N = 512
NK = 2 * (N - 1)
NKP = 1024
NSEG = 426
NSEGP = 512
NF = 16
NFH = 8
EPS = 1e-8
DUMP = N * N
QROWS = 128
QWORDS = QROWS * N

_f = np.fft.fftfreq(N).astype(np.float64)
_c = np.fft.fft(_f * _f).real
_i = np.arange(N)
_M_NP = _c[(_i[:, None] - _i[None, :]) % N].astype(np.float32)
_M_HI_NP = _M_NP.astype(np.dtype("bfloat16"))
_M_LO_NP = (_M_NP - _M_HI_NP.astype(np.float32)).astype(np.dtype("bfloat16"))


def _coords_body(p1x_r, p1y_r, p2x_r, p2y_r, segf_r, out_ref):
    p1x, p1y, p2x, p2y, segf = (
        p1x_r[0], p1y_r[0], p2x_r[0], p2y_r[0], segf_r[0],
    )
    x1 = jnp.floor(p1x * (N - 1)).astype(jnp.int32)
    y1 = jnp.floor(p1y * (N - 1)).astype(jnp.int32)
    x2 = jnp.floor(p2x * (N - 1)).astype(jnp.int32)
    y2 = jnp.floor(p2y * (N - 1)).astype(jnp.int32)
    in01 = (
        (p1x >= 0) & (p1x <= 1) & (p1y >= 0) & (p1y <= 1)
        & (p2x >= 0) & (p2x <= 1) & (p2y >= 0) & (p2y <= 1)
    )
    ok = in01 & (segf > 0)
    d = jnp.maximum(2 * jnp.maximum(jnp.abs(x2 - x1), jnp.abs(y2 - y1)), 2) - 1
    d2 = 2 * d
    rec = 1.0 / d2.astype(jnp.float32)
    k = lax.broadcasted_iota(jnp.int32, (NKP, NSEGP), 0)
    kk = jnp.minimum(k, d)

    def interp(a1, a2):
        num = 2 * (a1 * (d - kk) + a2 * kk) + d
        q0 = jnp.floor(num.astype(jnp.float32) * rec).astype(jnp.int32)
        r = num - q0 * d2
        return q0 + (r >= d2).astype(jnp.int32) - (r < 0).astype(jnp.int32)

    lx = interp(x1, x2)
    ly = interp(y1, y2)
    lin = ly * N + lx
    out_ref[0] = jnp.where(ok, lin, DUMP)


def _coords(p1x, p1y, p2x, p2y, segf):
    spec = pl.BlockSpec((1, 1, NSEGP), lambda i: (i, 0, 0))
    return pl.pallas_call(
        _coords_body,
        grid=(NFH,),
        in_specs=[spec] * 5,
        out_specs=pl.BlockSpec((1, NKP, NSEGP), lambda i: (i, 0, 0)),
        out_shape=jax.ShapeDtypeStruct((NFH, NKP, NSEGP), jnp.int32),
    )(p1x, p1y, p2x, p2y, segf)


KROWS = 32
NCH = NKP // KROWS
NSEG_SCAN = 432
JV = NSEG_SCAN // 16


def _scatter_fields(idx):
    mesh = plsc.VectorSubcoreMesh(core_axis_name="c", subcore_axis_name="s")

    @functools.partial(
        pl.kernel,
        mesh=mesh,
        out_type=jax.ShapeDtypeStruct((NFH, 4, QWORDS), jnp.float32),
        scratch_types=[
            pltpu.VMEM((KROWS, NSEGP), jnp.int32),
            pltpu.VMEM((KROWS, NSEGP), jnp.int32),
            pltpu.VMEM((QWORDS,), jnp.float32),
            pltpu.SemaphoreType.DMA,
            pltpu.SemaphoreType.DMA,
        ],
        compiler_params=pltpu.CompilerParams(needs_layout_passes=False),
    )
    def scat(idx_hbm, out_hbm, buf0, buf1, field, sem0, sem1):
        wid = lax.axis_index("s") * 2 + lax.axis_index("c")
        ones = jnp.full((16,), 1.0, jnp.float32)
        zeros16 = jnp.zeros((16,), jnp.float32)
        qwords_u = jnp.uint32(QWORDS)
        fld = wid // 4
        base = (wid % 4) * QWORDS

        def src(c):
            return idx_hbm.at[fld, pl.ds(c * KROWS, KROWS)]

        pltpu.async_copy(src(0), buf0, sem0)
        pltpu.async_copy(src(1), buf1, sem1)

        @plsc.parallel_loop(0, QWORDS // 16, unroll=8)
        def _(i):
            field[pl.ds(i * 16, 16)] = zeros16

        def do_chunk(c, buf, sem):
            pltpu.make_async_copy(src(c), buf, sem).wait()
            for r in range(KROWS):
                @plsc.parallel_loop(0, JV, unroll=3)
                def _(j):
                    idxv = buf[r, pl.ds(j * 16, 16)]
                    loc = idxv - base
                    msk = plsc.bitcast(loc, jnp.uint32) < qwords_u
                    plsc.store_scatter(field, [loc], ones, mask=msk)

            @pl.when(c + 2 < NCH)
            def _():
                pltpu.async_copy(src(c + 2), buf, sem)

        def pair_body(t, _):
            do_chunk(2 * t, buf0, sem0)
            do_chunk(2 * t + 1, buf1, sem1)
            return 0

        lax.fori_loop(0, NCH // 2, pair_body, 0)
        pltpu.sync_copy(field, out_hbm.at[fld, wid % 4])

    return scat(idx)


def _dilate3(x):
    z_row = jnp.zeros((1, N), jnp.float32)
    up = jnp.concatenate([x[1:], z_row], axis=0)
    dn = jnp.concatenate([z_row, x[:-1]], axis=0)
    v = jnp.maximum(x, jnp.maximum(up, dn))
    z_col = jnp.zeros((N, 1), jnp.float32)
    lf = jnp.concatenate([v[:, 1:], z_col], axis=1)
    rt = jnp.concatenate([z_col, v[:, :-1]], axis=1)
    return jnp.maximum(v, jnp.maximum(lf, rt))


def _loss_body(gt_ref, pr_ref, mhi_ref, out_ref):
    b = pl.program_id(0)
    g = _dilate3(gt_ref[0])
    p = _dilate3(pr_ref[0])
    dd = g - p
    ddb = dd.astype(jnp.bfloat16)
    mhi = mhi_ref[...]
    q = jnp.dot(mhi, ddb, preferred_element_type=jnp.float32) + jnp.dot(
        ddb, mhi, preferred_element_type=jnp.float32
    )
    part = jnp.float32(N) * jnp.sum(q * dd) + jnp.float32(EPS * N * N) * jnp.sum(dd * dd)
    part = part * jnp.float32(1.0 / (float(N) ** 4) / 8.0)

    @pl.when(b == 0)
    def _():
        out_ref[0, 0] = 0.0

    out_ref[0, 0] += part


def _spectral_loss(fields, mhi):
    return pl.pallas_call(
        _loss_body,
        grid=(NFH // 2,),
        in_specs=[
            pl.BlockSpec((1, N, N), lambda b: (2 * b, 0, 0)),
            pl.BlockSpec((1, N, N), lambda b: (2 * b + 1, 0, 0)),
            pl.BlockSpec((N, N), lambda b: (0, 0)),
        ],
        out_specs=pl.BlockSpec(memory_space=pltpu.SMEM),
        out_shape=jax.ShapeDtypeStruct((1, 1), jnp.float32),
    )(fields, fields, mhi)


def kernel(pred_keypoints, gt_keypoints, valid_mask):
    kp = jnp.stack([gt_keypoints, pred_keypoints], axis=1).reshape(NF, 6, 72, 2)
    vm = jnp.repeat(valid_mask, 2, axis=0)

    p1 = kp[:, :, :-1, :].reshape(NF, NSEG, 2)
    p2 = kp[:, :, 1:, :].reshape(NF, NSEG, 2)
    segv = vm[:, :, :-1] & vm[:, :, 1:] & jnp.any(vm, axis=2)[:, :, None]
    segf = segv.reshape(NF, NSEG).astype(jnp.float32)

    pad = NSEGP - NSEG

    def padded(a):
        return jnp.pad(a, ((0, 0), (0, pad)))[:, None, :]

    p1x = padded(p1[:, :, 0])
    p1y = padded(p1[:, :, 1])
    p2x = padded(p2[:, :, 0])
    p2y = padded(p2[:, :, 1])
    segf = padded(segf)

    mhi = jnp.asarray(_M_HI_NP)

    loss = jnp.float32(0.0)
    for h in range(2):
        sl = slice(h * NFH, (h + 1) * NFH)
        idx_h = _coords(p1x[sl], p1y[sl], p2x[sl], p2y[sl], segf[sl])
        quarters_h = _scatter_fields(idx_h)
        fields_h = quarters_h.reshape(NFH, N, N)
        out_h = _spectral_loss(fields_h, mhi)
        loss = loss + out_h[0, 0]
    return loss
